# Initial kernel scaffold; baseline (speedup 1.0000x reference)
import jax
import jax.numpy as jnp
from jax.experimental import pallas as pl


def kernel(emb_weight, edge_index, edge_type, w_rel1, w_root1, b1, w_rel2, w_root2, b2, w_pred, b_pred):
    raise NotImplementedError("write your pallas kernel here")


# speedup vs baseline: 2.5284x; 2.5284x over previous
import functools

import jax
import jax.numpy as jnp
from jax import lax
from jax.experimental import pallas as pl
from jax.experimental.pallas import tpu as pltpu
from jax.experimental.pallas import tpu_sc as plsc

N_NODES = 10000
HIDDEN = 128
N_REL = 2
N_USERS = 2000

NP = 10016
DUMMY = N_NODES
N_TILES = 16
T_ROWS = NP // N_TILES
ZCH = 313
HH = HIDDEN // 2
CHUNK = 128
BLK = 2048
RPB = BLK // CHUNK
EPT = 20480
NBLK = EPT // BLK
E_PAD = N_TILES * EPT


def _seg_sum_body(x0_hbm, x1_hbm, src_hbm, dst_hbm, typ_hbm, agg_out,
                  cnt_out, src_v, dst_v, typ_v, sidx_v, rows_v, rows_v2,
                  zero_v, cnt_loc, agg_sh, gsem, gsem2, ssem):
    c = lax.axis_index("c")
    s = lax.axis_index("s")
    z16 = jnp.zeros((16,), jnp.float32)
    one16 = jnp.ones((16,), jnp.float32)

    def zb(i, _):
        for k in range(HH // 16):
            zero_v[i, pl.ds(k * 16, 16)] = z16
        return 0
    lax.fori_loop(0, ZCH, zb, 0)

    def zc(i, _):
        cnt_loc[pl.ds(i * 16, 16)] = z16
        return 0
    lax.fori_loop(0, NP // 16, zc, 0)

    for h, xh_hbm in ((0, x0_hbm), (1, x1_hbm)):
        pltpu.sync_copy(zero_v, agg_sh.at[pl.ds(s * T_ROWS, ZCH)])
        pltpu.sync_copy(zero_v, agg_sh.at[pl.ds(s * T_ROWS + ZCH, ZCH)])
        plsc.subcore_barrier()

        def blk_body(b, _):
            base = (s * NBLK + b) * RPB
            pltpu.sync_copy(src_hbm.at[pl.ds(base, RPB)], src_v)
            pltpu.sync_copy(dst_hbm.at[pl.ds(base, RPB)], dst_v)
            pltpu.sync_copy(typ_hbm.at[pl.ds(base, RPB)], typ_v)

            def cidx(j, _):
                for k in range(CHUNK // 16):
                    t = typ_v[j, pl.ds(k * 16, 16)]
                    d = dst_v[j, pl.ds(k * 16, 16)]
                    idx = jnp.where(t == c, d, DUMMY)
                    sidx_v[j, pl.ds(k * 16, 16)] = idx
                    if h == 0:
                        plsc.addupdate_scatter(cnt_loc, [idx], one16)
                return 0
            lax.fori_loop(0, RPB, cidx, 0)

            def pair(p, _):
                g0 = pltpu.async_copy(xh_hbm.at[src_v.at[2 * p]], rows_v,
                                      gsem)
                g1 = pltpu.async_copy(xh_hbm.at[src_v.at[2 * p + 1]],
                                      rows_v2, gsem2)
                g0.wait()
                pltpu.async_copy(rows_v, agg_sh.at[sidx_v.at[2 * p]], ssem,
                                 add=True).wait()
                g1.wait()
                pltpu.async_copy(rows_v2, agg_sh.at[sidx_v.at[2 * p + 1]],
                                 ssem, add=True).wait()
                return 0
            lax.fori_loop(0, RPB // 2, pair, 0)
            return 0
        lax.fori_loop(0, NBLK, blk_body, 0)

        if h == 0:
            pltpu.sync_copy(cnt_loc, cnt_out.at[c, s])
        plsc.subcore_barrier()

        pltpu.sync_copy(agg_sh.at[pl.ds(s * T_ROWS, T_ROWS)],
                        agg_out.at[c, h, pl.ds(s * T_ROWS, T_ROWS)])
        plsc.subcore_barrier()


@jax.jit
def _seg_sum(x0, x1, src2d, dst2d, typ2d):
    mesh = plsc.VectorSubcoreMesh(core_axis_name="c", subcore_axis_name="s",
                                  num_cores=N_REL, num_subcores=N_TILES)
    f = pl.kernel(
        _seg_sum_body,
        out_type=(
            jax.ShapeDtypeStruct((N_REL, 2, NP, HH), jnp.float32),
            jax.ShapeDtypeStruct((N_REL, N_TILES, NP), jnp.float32),
        ),
        mesh=mesh,
        compiler_params=pltpu.CompilerParams(needs_layout_passes=False,
                                             use_tc_tiling_on_sc=False),
        scratch_types=[
            pltpu.VMEM((RPB, CHUNK), jnp.int32),
            pltpu.VMEM((RPB, CHUNK), jnp.int32),
            pltpu.VMEM((RPB, CHUNK), jnp.int32),
            pltpu.VMEM((RPB, CHUNK), jnp.int32),
            pltpu.VMEM((CHUNK, HH), jnp.float32),
            pltpu.VMEM((CHUNK, HH), jnp.float32),
            pltpu.VMEM((ZCH, HH), jnp.float32),
            pltpu.VMEM((NP,), jnp.float32),
            pltpu.VMEM_SHARED((NP, HH), jnp.float32),
            pltpu.SemaphoreType.DMA,
            pltpu.SemaphoreType.DMA,
            pltpu.SemaphoreType.DMA,
        ],
    )
    return f(x0, x1, src2d, dst2d, typ2d)


_ROWB = 1000


def _combine_body(x_ref, a0_ref, a1_ref, c0_ref, c1_ref, wr_ref, w0_ref,
                  w1_ref, b_ref, out_ref):
    inv0 = (1.0 / jnp.maximum(jnp.sum(c0_ref[...], axis=1), 1.0))[:, None]
    inv1 = (1.0 / jnp.maximum(jnp.sum(c1_ref[...], axis=1), 1.0))[:, None]
    acc = jnp.dot(x_ref[...], wr_ref[...], preferred_element_type=jnp.float32)
    acc += jnp.dot(a0_ref[...] * inv0, w0_ref[...],
                   preferred_element_type=jnp.float32)
    acc += jnp.dot(a1_ref[...] * inv1, w1_ref[...],
                   preferred_element_type=jnp.float32)
    out_ref[...] = acc + b_ref[...]


def _pred_body(x_ref, wp_ref, bp_ref, out_ref):
    out_ref[...] = (jnp.dot(x_ref[...], wp_ref[...],
                            preferred_element_type=jnp.float32)
                    + bp_ref[...])


def _row_spec():
    return pl.BlockSpec((_ROWB, HIDDEN), lambda i: (i, 0))


def _cnt_spec():
    return pl.BlockSpec((_ROWB, N_TILES), lambda i: (i, 0))


def _full_spec(shape):
    return pl.BlockSpec(shape, lambda i: tuple(0 for _ in shape))


@jax.jit
def _combine(x, a0, a1, c0, c1, wr, w0, w1, b):
    grid = (N_NODES // _ROWB,)
    return pl.pallas_call(
        _combine_body,
        grid=grid,
        in_specs=[
            _row_spec(), _row_spec(), _row_spec(), _cnt_spec(), _cnt_spec(),
            _full_spec((HIDDEN, HIDDEN)), _full_spec((HIDDEN, HIDDEN)),
            _full_spec((HIDDEN, HIDDEN)), _full_spec((1, HIDDEN)),
        ],
        out_specs=_row_spec(),
        out_shape=jax.ShapeDtypeStruct((N_NODES, HIDDEN), jnp.float32),
        compiler_params=pltpu.CompilerParams(
            dimension_semantics=("parallel",)),
    )(x, a0, a1, c0, c1, wr, w0, w1, b)


@jax.jit
def _pred(x, wp, bp):
    grid = (N_NODES // _ROWB,)
    return pl.pallas_call(
        _pred_body,
        grid=grid,
        in_specs=[
            _row_spec(), _full_spec((HIDDEN, N_USERS)),
            _full_spec((1, N_USERS)),
        ],
        out_specs=pl.BlockSpec((_ROWB, N_USERS), lambda i: (i, 0)),
        out_shape=jax.ShapeDtypeStruct((N_NODES, N_USERS), jnp.float32),
        compiler_params=pltpu.CompilerParams(
            dimension_semantics=("parallel",)),
    )(x, wp, bp)


def kernel(emb_weight, edge_index, edge_type, w_rel1, w_root1, b1,
           w_rel2, w_root2, b2, w_pred, b_pred):
    e = edge_index.shape[1]
    pad = E_PAD - e
    src2d = jnp.pad(edge_index[0], (0, pad)).reshape(E_PAD // CHUNK, CHUNK)
    dst2d = jnp.pad(edge_index[1], (0, pad)).reshape(E_PAD // CHUNK, CHUNK)
    typ2d = jnp.pad(edge_type, (0, pad),
                    constant_values=N_REL).reshape(E_PAD // CHUNK, CHUNK)

    w_root_s = jnp.stack([w_root1, w_root2])
    w_rel_s = jnp.stack([w_rel1, w_rel2])
    b_s = jnp.stack([b1, b2])

    def layer(i, x):
        agg, cnt = _seg_sum(x[:, :HH], x[:, HH:], src2d, dst2d, typ2d)
        a0 = jnp.concatenate([agg[0, 0, :N_NODES], agg[0, 1, :N_NODES]],
                             axis=1)
        a1 = jnp.concatenate([agg[1, 0, :N_NODES], agg[1, 1, :N_NODES]],
                             axis=1)
        return _combine(x, a0, a1, cnt[0].T, cnt[1].T, w_root_s[i],
                        w_rel_s[i, 0], w_rel_s[i, 1], b_s[i][None])

    h2 = lax.fori_loop(0, 2, layer, emb_weight)
    return _pred(h2, w_pred, b_pred[None])


# speedup vs baseline: 5.5696x; 2.2028x over previous
import functools

import jax
import jax.numpy as jnp
from jax import lax
from jax.experimental import pallas as pl
from jax.experimental.pallas import tpu as pltpu
from jax.experimental.pallas import tpu_sc as plsc

N_NODES = 10000
HIDDEN = 128
N_REL = 2
N_USERS = 2000

NP = 10016
DUMMY = N_NODES
N_TILES = 16
T_ROWS = NP // N_TILES
ZCH = 313
HH = HIDDEN // 2
CHUNK = 128
BLK = 2048
RPB = BLK // CHUNK
EPT = 20480
NBLK = EPT // BLK
E_PAD = N_TILES * EPT


def _seg_sum_body(x0_hbm, x1_hbm, pk_hbm, agg_out,
                  cnt_out, pk_v, cpk, sstage,
                  rows_v, rows_v2, zero_v, cnt_loc, agg_sh, gsem, gsem2,
                  ssem):
    c = lax.axis_index("c")
    s = lax.axis_index("s")
    z16 = jnp.zeros((16,), jnp.float32)
    one16 = jnp.ones((16,), jnp.float32)

    def zb(i, _):
        for k in range(HH // 16):
            zero_v[i, pl.ds(k * 16, 16)] = z16
        return 0
    lax.fori_loop(0, ZCH, zb, 0)

    def zc(i, _):
        cnt_loc[pl.ds(i * 16, 16)] = z16
        return 0
    lax.fori_loop(0, NP // 16, zc, 0)

    def blk_body(b, off):
        base = (s * NBLK + b) * RPB
        pltpu.sync_copy(pk_hbm.at[pl.ds(base, RPB)], pk_v)

        def row(j, off):
            for k in range(CHUNK // 16):
                pk = pk_v[j, pl.ds(k * 16, 16)]
                d = pk & 16383
                m = (pk >> 28) == c
                plsc.addupdate_scatter(cnt_loc, [jnp.where(m, d, DUMMY)],
                                       one16)
                plsc.store_compressed(cpk.at[pl.ds(off, 16)], pk, mask=m)
                pc = lax.reduce_max(plsc.all_reduce_population_count(m),
                                    axes=(0,))
                off = off + pc
            return off
        return lax.fori_loop(0, RPB, row, off)
    off = lax.fori_loop(0, NBLK, blk_body, 0)

    def fill(i, off):
        cpk[pl.ds(off, 16)] = jnp.full((16,), DUMMY, jnp.int32)
        return off + jnp.where(off % (2 * CHUNK) != 0, 16, 0)
    off = lax.fori_loop(0, 2 * CHUNK // 16, fill, off)
    npairs = off // (2 * CHUNK)

    pltpu.sync_copy(cnt_loc, cnt_out.at[c, s])

    for h, xh_hbm in ((0, x0_hbm), (1, x1_hbm)):
        pltpu.sync_copy(zero_v, agg_sh.at[pl.ds(s * T_ROWS, ZCH)])
        pltpu.sync_copy(zero_v, agg_sh.at[pl.ds(s * T_ROWS + ZCH, ZCH)])
        plsc.subcore_barrier()

        def pair(p, _):
            for q in range(CHUNK // 16):
                pk0 = cpk[pl.ds(p * 2 * CHUNK + q * 16, 16)]
                pk1 = cpk[pl.ds(p * 2 * CHUNK + CHUNK + q * 16, 16)]
                sstage[0, pl.ds(q * 16, 16)] = pk0 & 16383
                sstage[1, pl.ds(q * 16, 16)] = pk1 & 16383
                sstage[2, pl.ds(q * 16, 16)] = (pk0 >> 14) & 16383
                sstage[3, pl.ds(q * 16, 16)] = (pk1 >> 14) & 16383
            g0 = pltpu.async_copy(xh_hbm.at[sstage.at[2]], rows_v, gsem)
            g1 = pltpu.async_copy(xh_hbm.at[sstage.at[3]], rows_v2, gsem2)
            g0.wait()
            pltpu.async_copy(rows_v, agg_sh.at[sstage.at[0]], ssem,
                             add=True).wait()
            g1.wait()
            pltpu.async_copy(rows_v2, agg_sh.at[sstage.at[1]], ssem,
                             add=True).wait()
            return 0
        lax.fori_loop(0, npairs, pair, 0)
        plsc.subcore_barrier()

        pltpu.sync_copy(agg_sh.at[pl.ds(s * T_ROWS, T_ROWS)],
                        agg_out.at[c, h, pl.ds(s * T_ROWS, T_ROWS)])
        plsc.subcore_barrier()


@jax.jit
def _seg_sum(x0, x1, pk2d):
    mesh = plsc.VectorSubcoreMesh(core_axis_name="c", subcore_axis_name="s",
                                  num_cores=N_REL, num_subcores=N_TILES)
    f = pl.kernel(
        _seg_sum_body,
        out_type=(
            jax.ShapeDtypeStruct((N_REL, 2, NP, HH), jnp.float32),
            jax.ShapeDtypeStruct((N_REL, N_TILES, NP), jnp.float32),
        ),
        mesh=mesh,
        compiler_params=pltpu.CompilerParams(needs_layout_passes=False,
                                             use_tc_tiling_on_sc=False),
        scratch_types=[
            pltpu.VMEM((RPB, CHUNK), jnp.int32),
            pltpu.VMEM((EPT + 2 * CHUNK,), jnp.int32),
            pltpu.VMEM((4, CHUNK), jnp.int32),
            pltpu.VMEM((CHUNK, HH), jnp.float32),
            pltpu.VMEM((CHUNK, HH), jnp.float32),
            pltpu.VMEM((ZCH, HH), jnp.float32),
            pltpu.VMEM((NP,), jnp.float32),
            pltpu.VMEM_SHARED((NP, HH), jnp.float32),
            pltpu.SemaphoreType.DMA,
            pltpu.SemaphoreType.DMA,
            pltpu.SemaphoreType.DMA,
        ],
    )
    return f(x0, x1, pk2d)


_ROWB = 1000


def _combine_body(x_ref, a0_ref, a1_ref, c0_ref, c1_ref, wr_ref, w0_ref,
                  w1_ref, b_ref, out_ref):
    inv0 = (1.0 / jnp.maximum(jnp.sum(c0_ref[...], axis=1), 1.0))[:, None]
    inv1 = (1.0 / jnp.maximum(jnp.sum(c1_ref[...], axis=1), 1.0))[:, None]
    acc = jnp.dot(x_ref[...], wr_ref[...], preferred_element_type=jnp.float32)
    acc += jnp.dot(a0_ref[...] * inv0, w0_ref[...],
                   preferred_element_type=jnp.float32)
    acc += jnp.dot(a1_ref[...] * inv1, w1_ref[...],
                   preferred_element_type=jnp.float32)
    out_ref[...] = acc + b_ref[...]


def _pred_body(x_ref, wp_ref, bp_ref, out_ref):
    out_ref[...] = (jnp.dot(x_ref[...], wp_ref[...],
                            preferred_element_type=jnp.float32)
                    + bp_ref[...])


def _row_spec():
    return pl.BlockSpec((_ROWB, HIDDEN), lambda i: (i, 0))


def _cnt_spec():
    return pl.BlockSpec((_ROWB, N_TILES), lambda i: (i, 0))


def _full_spec(shape):
    return pl.BlockSpec(shape, lambda i: tuple(0 for _ in shape))


@jax.jit
def _combine(x, a0, a1, c0, c1, wr, w0, w1, b):
    grid = (N_NODES // _ROWB,)
    return pl.pallas_call(
        _combine_body,
        grid=grid,
        in_specs=[
            _row_spec(), _row_spec(), _row_spec(), _cnt_spec(), _cnt_spec(),
            _full_spec((HIDDEN, HIDDEN)), _full_spec((HIDDEN, HIDDEN)),
            _full_spec((HIDDEN, HIDDEN)), _full_spec((1, HIDDEN)),
        ],
        out_specs=_row_spec(),
        out_shape=jax.ShapeDtypeStruct((N_NODES, HIDDEN), jnp.float32),
        compiler_params=pltpu.CompilerParams(
            dimension_semantics=("parallel",)),
    )(x, a0, a1, c0, c1, wr, w0, w1, b)


@jax.jit
def _pred(x, wp, bp):
    grid = (N_NODES // _ROWB,)
    return pl.pallas_call(
        _pred_body,
        grid=grid,
        in_specs=[
            _row_spec(), _full_spec((HIDDEN, N_USERS)),
            _full_spec((1, N_USERS)),
        ],
        out_specs=pl.BlockSpec((_ROWB, N_USERS), lambda i: (i, 0)),
        out_shape=jax.ShapeDtypeStruct((N_NODES, N_USERS), jnp.float32),
        compiler_params=pltpu.CompilerParams(
            dimension_semantics=("parallel",)),
    )(x, wp, bp)


def kernel(emb_weight, edge_index, edge_type, w_rel1, w_root1, b1,
           w_rel2, w_root2, b2, w_pred, b_pred):
    e = edge_index.shape[1]
    pad = E_PAD - e
    pk = ((edge_type.astype(jnp.int32) << 28)
          | (edge_index[0].astype(jnp.int32) << 14)
          | edge_index[1].astype(jnp.int32))
    pk2d = jnp.pad(pk, (0, pad),
                   constant_values=N_REL << 28).reshape(E_PAD // CHUNK, CHUNK)

    w_root_s = jnp.stack([w_root1, w_root2])
    w_rel_s = jnp.stack([w_rel1, w_rel2])
    b_s = jnp.stack([b1, b2])

    def layer(i, x):
        agg, cnt = _seg_sum(x[:, :HH], x[:, HH:], pk2d)
        a0 = jnp.concatenate([agg[0, 0, :N_NODES], agg[0, 1, :N_NODES]],
                             axis=1)
        a1 = jnp.concatenate([agg[1, 0, :N_NODES], agg[1, 1, :N_NODES]],
                             axis=1)
        return _combine(x, a0, a1, cnt[0].T, cnt[1].T, w_root_s[i],
                        w_rel_s[i, 0], w_rel_s[i, 1], b_s[i][None])

    h2 = lax.fori_loop(0, 2, layer, emb_weight)
    return _pred(h2, w_pred, b_pred[None])
